# baseline (speedup 1.0000x reference)
import functools

import jax
import jax.numpy as jnp
from jax import lax
from jax.experimental import pallas as pl
from jax.experimental.pallas import tpu as pltpu
from jax.experimental.pallas import tpu_sc as plsc

_B = 2
_S = 8192
_D = 1024
_NC = 2
_NS = 16
_NW = _NC * _NS
_ROWS_PER_W = _S // _NW
_R = 8
_NCHUNK = _ROWS_PER_W // _R
_NBUF = 4
_VPR = _D // 16

_mesh = plsc.VectorSubcoreMesh(core_axis_name="c", subcore_axis_name="s")


@functools.partial(
    pl.kernel,
    mesh=_mesh,
    out_type=jax.ShapeDtypeStruct((_B, _S, _D), jnp.float32),
    scratch_types=[
        [pltpu.VMEM((_R, _D), jnp.float32)] * _NBUF,
        [pltpu.VMEM((_R, _D), jnp.float32)] * _NBUF,
        [pltpu.VMEM((_R, _D), jnp.float32)] * _NBUF,
        [pltpu.SemaphoreType.DMA] * _NBUF,
        [pltpu.SemaphoreType.DMA] * _NBUF,
    ],
)
def _sc_add(x_hbm, pos_hbm, out_hbm, pos_v, x0_v, x1_v, sin, sout):
    wid = lax.axis_index("s") * _NC + lax.axis_index("c")
    base = wid * _ROWS_PER_W

    def start_in(g):
        b = g % _NBUF
        row0 = base + g * _R
        return (
            pltpu.async_copy(pos_hbm.at[pl.ds(row0, _R)], pos_v[b], sin[b]),
            pltpu.async_copy(x_hbm.at[0, pl.ds(row0, _R)], x0_v[b], sin[b]),
            pltpu.async_copy(x_hbm.at[1, pl.ds(row0, _R)], x1_v[b], sin[b]),
        )

    def start_out(g):
        b = g % _NBUF
        row0 = base + g * _R
        return (
            pltpu.async_copy(x0_v[b], out_hbm.at[0, pl.ds(row0, _R)], sout[b]),
            pltpu.async_copy(x1_v[b], out_hbm.at[1, pl.ds(row0, _R)], sout[b]),
        )

    in_descs = [None] * _NCHUNK
    out_descs = [None] * _NCHUNK
    in_descs[0] = start_in(0)
    in_descs[1] = start_in(1)

    for g in range(_NCHUNK):
        b = g % _NBUF
        for d in in_descs[g]:
            d.wait()

        @plsc.parallel_loop(0, _VPR, unroll=2)
        def _vec(i):
            sl = pl.ds(i * 16, 16)
            for r in range(_R):
                p = pos_v[b][r, sl]
                x0_v[b][r, sl] = x0_v[b][r, sl] + p
                x1_v[b][r, sl] = x1_v[b][r, sl] + p

        out_descs[g] = start_out(g)
        nxt = g + 2
        if nxt < _NCHUNK:
            if nxt - _NBUF >= 0:
                for d in out_descs[nxt - _NBUF]:
                    d.wait()
            in_descs[nxt] = start_in(nxt)

    for g in range(_NCHUNK - _NBUF, _NCHUNK):
        for d in out_descs[g]:
            d.wait()


def kernel(x, pos_table):
    return _sc_add(x, pos_table)


# speedup vs baseline: 1.0800x; 1.0800x over previous
import functools

import jax
import jax.numpy as jnp
from jax import lax
from jax.experimental import pallas as pl
from jax.experimental.pallas import tpu as pltpu
from jax.experimental.pallas import tpu_sc as plsc

_B = 2
_S = 8192
_D = 1024
_NC = 2
_NS = 16
_NW = _NC * _NS
_ROWS_PER_W = _S // _NW
_R = 8
_NCHUNK = _ROWS_PER_W // _R
_NBUF = 4
_VPR = _D // 16

_mesh = plsc.VectorSubcoreMesh(core_axis_name="c", subcore_axis_name="s")


@functools.partial(
    pl.kernel,
    mesh=_mesh,
    out_type=jax.ShapeDtypeStruct((_B, _S, _D), jnp.float32),
    scratch_types=[
        [pltpu.VMEM((_R, _D), jnp.float32)] * _NBUF,
        [pltpu.VMEM((_R, _D), jnp.float32)] * _NBUF,
        [pltpu.VMEM((_R, _D), jnp.float32)] * _NBUF,
        [pltpu.SemaphoreType.DMA] * _NBUF,
        [pltpu.SemaphoreType.DMA] * _NBUF,
    ],
)
def _sc_add(x_hbm, pos_hbm, out_hbm, pos_v, x0_v, x1_v, sin, sout):
    wid = lax.axis_index("s") * _NC + lax.axis_index("c")
    base = wid * _ROWS_PER_W

    def start_in(g):
        b = g % _NBUF
        row0 = base + g * _R
        return (
            pltpu.async_copy(pos_hbm.at[pl.ds(row0, _R)], pos_v[b], sin[b]),
            pltpu.async_copy(x_hbm.at[0, pl.ds(row0, _R)], x0_v[b], sin[b]),
            pltpu.async_copy(x_hbm.at[1, pl.ds(row0, _R)], x1_v[b], sin[b]),
        )

    def start_out(g):
        b = g % _NBUF
        row0 = base + g * _R
        return (
            pltpu.async_copy(x0_v[b], out_hbm.at[0, pl.ds(row0, _R)], sout[b]),
            pltpu.async_copy(x1_v[b], out_hbm.at[1, pl.ds(row0, _R)], sout[b]),
        )

    in_descs = [None] * _NCHUNK
    out_descs = [None] * _NCHUNK
    in_descs[0] = start_in(0)
    in_descs[1] = start_in(1)

    for g in range(_NCHUNK):
        b = g % _NBUF
        for d in in_descs[g]:
            d.wait()

        @plsc.parallel_loop(0, _R * _VPR, unroll=8)
        def _vec(i):
            r = i // _VPR
            sl = pl.ds((i % _VPR) * 16, 16)
            p = pos_v[b][r, sl]
            x0_v[b][r, sl] = x0_v[b][r, sl] + p
            x1_v[b][r, sl] = x1_v[b][r, sl] + p

        out_descs[g] = start_out(g)
        nxt = g + 2
        if nxt < _NCHUNK:
            if nxt - _NBUF >= 0:
                for d in out_descs[nxt - _NBUF]:
                    d.wait()
            in_descs[nxt] = start_in(nxt)

    for g in range(_NCHUNK - _NBUF, _NCHUNK):
        for d in out_descs[g]:
            d.wait()


def kernel(x, pos_table):
    return _sc_add(x, pos_table)
